# baseline (speedup 1.0000x reference)
import functools

import jax
import jax.numpy as jnp
from jax import lax
from jax.experimental import pallas as pl
from jax.experimental.pallas import tpu as pltpu
from jax.experimental.pallas import tpu_sc as plsc

BATCH = 4096
SEQ = 200
SEQ_PAD = 256
EMBED = 64
CHUNK = 512
NCHUNK = BATCH // CHUNK
SEQ_SPLITS = ((0, 104), (104, 96))


def _permute_body(s_ref, ps_ref, lrow_ref, rank_ref):
    f32 = jnp.float32
    s = s_ref[...]
    mask = (s > 0).astype(f32)
    lcol = jnp.sum(mask, axis=1, keepdims=True)

    i0 = lax.broadcasted_iota(jnp.int32, (CHUNK, CHUNK), 0)
    i1 = lax.broadcasted_iota(jnp.int32, (CHUNK, CHUNK), 1)
    eye = (i0 == i1).astype(f32)
    for c in range(NCHUNK):
        lc = lcol[c * CHUNK:(c + 1) * CHUNK, :]
        lrow_ref[:, c * CHUNK:(c + 1) * CHUNK] = lax.dot_general(
            lc, eye, (((0,), (0,)), ((), ())))
    lrow = lrow_ref[...]

    jj = lax.broadcasted_iota(jnp.int32, (CHUNK, BATCH), 1)
    ii = lax.broadcasted_iota(jnp.int32, (CHUNK, BATCH), 0)
    for c in range(NCHUNK):
        li = lcol[c * CHUNK:(c + 1) * CHUNK, :]
        gi = ii + c * CHUNK
        gt = (lrow > li).astype(f32)
        tie = ((lrow == li) & (jj < gi)).astype(f32)
        rank_ref[c * CHUNK:(c + 1) * CHUNK, :] = jnp.sum(
            gt + tie, axis=1, keepdims=True)

    jjf = jj.astype(f32)
    ps = jnp.zeros((BATCH, SEQ_PAD), f32)
    dims = (((0,), (0,)), ((), ()))
    for c in range(NCHUNK):
        rc = rank_ref[c * CHUNK:(c + 1) * CHUNK, :]
        match = (rc == jjf).astype(f32)
        sc_rows = s[c * CHUNK:(c + 1) * CHUNK, :]
        pa = lax.dot_general(
            match, (sc_rows >> 16).astype(f32), dims)
        pb = lax.dot_general(
            match, ((sc_rows >> 8) & 255).astype(f32), dims)
        pc = lax.dot_general(
            match, (sc_rows & 255).astype(f32), dims)
        ps = ps + (pa * 65536.0 + pb * 256.0 + pc)
    ps_ref[...] = ps[:, :SEQ].astype(jnp.int32)


def _tc_permute(samples_padded):
    return pl.pallas_call(
        _permute_body,
        out_shape=jax.ShapeDtypeStruct((BATCH, SEQ), jnp.int32),
        scratch_shapes=[
            pltpu.VMEM((1, BATCH), jnp.float32),
            pltpu.VMEM((BATCH, 1), jnp.float32),
        ],
    )(samples_padded)


ROWS_PER_STAGE = 2


def _sc_gather(ps_flat, table):
    info = plsc.get_sparse_core_info()
    nc, ns = info.num_cores, info.num_subcores
    nw = nc * ns
    rpw = BATCH // nw
    nstage = rpw // ROWS_PER_STAGE
    mesh = plsc.VectorSubcoreMesh(core_axis_name="c", subcore_axis_name="s")

    @functools.partial(
        pl.kernel,
        mesh=mesh,
        out_type=jax.ShapeDtypeStruct((BATCH, SEQ, EMBED), jnp.float32),
        scratch_types=[
            pltpu.VMEM((rpw * SEQ,), jnp.int32),
            pltpu.VMEM((ROWS_PER_STAGE, SEQ, EMBED), jnp.float32),
            pltpu.VMEM((ROWS_PER_STAGE, SEQ, EMBED), jnp.float32),
            pltpu.SemaphoreType.DMA,
            pltpu.SemaphoreType.DMA,
            pltpu.SemaphoreType.DMA,
            pltpu.SemaphoreType.DMA,
        ],
        compiler_params=pltpu.CompilerParams(use_tc_tiling_on_sc=False),
    )
    def k(ps_hbm, table_hbm, out_hbm, ids_v, s0, s1, g0, g1, w0, w1):
        wid = lax.axis_index("s") * nc + lax.axis_index("c")
        r0 = wid * rpw
        pltpu.sync_copy(ps_hbm.at[pl.ds(r0 * SEQ, rpw * SEQ)], ids_v)

        def issue_gathers(r, buf, gsem):
            for j in range(ROWS_PER_STAGE):
                for off, n in SEQ_SPLITS:
                    pltpu.async_copy(
                        table_hbm.at[
                            ids_v.at[pl.ds((r * ROWS_PER_STAGE + j) * SEQ + off, n)]],
                        buf.at[j, pl.ds(off, n)], gsem)

        def wait_gathers(buf, gsem):
            for j in range(ROWS_PER_STAGE):
                for off, n in SEQ_SPLITS:
                    pltpu.make_async_copy(
                        table_hbm.at[ids_v.at[pl.ds(off, n)]],
                        buf.at[j, pl.ds(off, n)], gsem).wait()

        def issue_write(r, buf, wsem):
            pltpu.async_copy(
                buf, out_hbm.at[pl.ds(r0 + r * ROWS_PER_STAGE, ROWS_PER_STAGE)],
                wsem)

        def wait_write(r, buf, wsem):
            pltpu.make_async_copy(
                buf, out_hbm.at[pl.ds(r0 + r * ROWS_PER_STAGE, ROWS_PER_STAGE)],
                wsem).wait()

        issue_gathers(0, s0, g0)
        issue_gathers(1, s1, g1)

        def body(kk, carry):
            r = 2 * kk
            wait_gathers(s0, g0)
            issue_write(r, s0, w0)
            wait_gathers(s1, g1)
            issue_write(r + 1, s1, w1)

            @pl.when(r + 2 < nstage)
            def _():
                wait_write(r, s0, w0)
                issue_gathers(r + 2, s0, g0)

            @pl.when(r + 3 < nstage)
            def _():
                wait_write(r + 1, s1, w1)
                issue_gathers(r + 3, s1, g1)

            return carry

        lax.fori_loop(0, nstage // 2, body, 0)
        wait_write(nstage - 2, s0, w0)
        wait_write(nstage - 1, s1, w1)

    return k(ps_flat, table)


def kernel(samples, table):
    s32 = samples.astype(jnp.int32)
    sp = jnp.pad(s32, ((0, 0), (0, SEQ_PAD - SEQ)))
    ps = _tc_permute(sp)
    return _sc_gather(ps.reshape(BATCH * SEQ), table)


# speedup vs baseline: 1.0077x; 1.0077x over previous
import functools

import jax
import jax.numpy as jnp
from jax import lax
from jax.experimental import pallas as pl
from jax.experimental.pallas import tpu as pltpu
from jax.experimental.pallas import tpu_sc as plsc

BATCH = 4096
SEQ = 200
SEQ_PAD = 256
EMBED = 64
CHUNK = 512
NCHUNK = BATCH // CHUNK
SEQ_SPLITS = ((0, 104), (104, 96))


def _permute_body(s_ref, ps_ref, lrow_ref, rank_ref):
    f32 = jnp.float32
    s = s_ref[...]
    mask = (s > 0).astype(f32)
    lcol = jnp.sum(mask, axis=1, keepdims=True)

    i0 = lax.broadcasted_iota(jnp.int32, (CHUNK, CHUNK), 0)
    i1 = lax.broadcasted_iota(jnp.int32, (CHUNK, CHUNK), 1)
    eye = (i0 == i1).astype(f32)
    for c in range(NCHUNK):
        lc = lcol[c * CHUNK:(c + 1) * CHUNK, :]
        lrow_ref[:, c * CHUNK:(c + 1) * CHUNK] = lax.dot_general(
            lc, eye, (((0,), (0,)), ((), ())))
    lrow = lrow_ref[...]

    jj = lax.broadcasted_iota(jnp.int32, (CHUNK, BATCH), 1)
    ii = lax.broadcasted_iota(jnp.int32, (CHUNK, BATCH), 0)
    for c in range(NCHUNK):
        li = lcol[c * CHUNK:(c + 1) * CHUNK, :]
        gi = ii + c * CHUNK
        gt = (lrow > li).astype(f32)
        tie = ((lrow == li) & (jj < gi)).astype(f32)
        rank_ref[c * CHUNK:(c + 1) * CHUNK, :] = jnp.sum(
            gt + tie, axis=1, keepdims=True)

    jjf = jj.astype(f32)
    ps = jnp.zeros((BATCH, SEQ_PAD), f32)
    dims = (((0,), (0,)), ((), ()))
    for c in range(NCHUNK):
        rc = rank_ref[c * CHUNK:(c + 1) * CHUNK, :]
        match = (rc == jjf).astype(f32)
        sc_rows = s[c * CHUNK:(c + 1) * CHUNK, :]
        pa = lax.dot_general(
            match, (sc_rows >> 16).astype(f32), dims)
        pb = lax.dot_general(
            match, ((sc_rows >> 8) & 255).astype(f32), dims)
        pc = lax.dot_general(
            match, (sc_rows & 255).astype(f32), dims)
        ps = ps + (pa * 65536.0 + pb * 256.0 + pc)
    ps_ref[...] = ps[:, :SEQ].astype(jnp.int32)


def _tc_permute(samples_padded):
    return pl.pallas_call(
        _permute_body,
        out_shape=jax.ShapeDtypeStruct((BATCH, SEQ), jnp.int32),
        scratch_shapes=[
            pltpu.VMEM((1, BATCH), jnp.float32),
            pltpu.VMEM((BATCH, 1), jnp.float32),
        ],
    )(samples_padded)


NBUF = 4


def _sc_gather(ps_flat, table):
    info = plsc.get_sparse_core_info()
    nc, ns = info.num_cores, info.num_subcores
    nw = nc * ns
    rpw = BATCH // nw
    mesh = plsc.VectorSubcoreMesh(core_axis_name="c", subcore_axis_name="s")

    @functools.partial(
        pl.kernel,
        mesh=mesh,
        out_type=jax.ShapeDtypeStruct((BATCH, SEQ, EMBED), jnp.float32),
        scratch_types=[
            pltpu.VMEM((rpw * SEQ,), jnp.int32),
            [pltpu.VMEM((1, SEQ, EMBED), jnp.float32) for _ in range(NBUF)],
            [pltpu.SemaphoreType.DMA for _ in range(NBUF)],
            [pltpu.SemaphoreType.DMA for _ in range(NBUF)],
        ],
        compiler_params=pltpu.CompilerParams(use_tc_tiling_on_sc=False),
    )
    def k(ps_hbm, table_hbm, out_hbm, ids_v, bufs, gsems, wsems):
        wid = lax.axis_index("s") * nc + lax.axis_index("c")
        r0 = wid * rpw
        pltpu.sync_copy(ps_hbm.at[pl.ds(r0 * SEQ, rpw * SEQ)], ids_v)

        def issue_gathers(r, buf, gsem):
            for off, n in SEQ_SPLITS:
                pltpu.async_copy(
                    table_hbm.at[ids_v.at[pl.ds(r * SEQ + off, n)]],
                    buf.at[0, pl.ds(off, n)], gsem)

        def wait_gathers(buf, gsem):
            for off, n in SEQ_SPLITS:
                pltpu.make_async_copy(
                    table_hbm.at[ids_v.at[pl.ds(off, n)]],
                    buf.at[0, pl.ds(off, n)], gsem).wait()

        def issue_write(r, buf, wsem):
            pltpu.async_copy(buf, out_hbm.at[pl.ds(r0 + r, 1)], wsem)

        def wait_write(buf, wsem):
            pltpu.make_async_copy(
                buf, out_hbm.at[pl.ds(r0, 1)], wsem).wait()

        for b in range(NBUF):
            issue_gathers(b, bufs[b], gsems[b])

        def body(kk, carry):
            r = NBUF * kk
            for b in range(NBUF):
                wait_gathers(bufs[b], gsems[b])
                issue_write(r + b, bufs[b], wsems[b])

            for b in range(NBUF):
                @pl.when(r + NBUF + b < rpw)
                def _(b=b):
                    wait_write(bufs[b], wsems[b])
                    issue_gathers(r + NBUF + b, bufs[b], gsems[b])

            return carry

        lax.fori_loop(0, rpw // NBUF, body, 0)
        for b in range(NBUF):
            wait_write(bufs[b], wsems[b])

    return k(ps_flat, table)


def kernel(samples, table):
    s32 = samples.astype(jnp.int32)
    sp = jnp.pad(s32, ((0, 0), (0, SEQ_PAD - SEQ)))
    ps = _tc_permute(sp)
    return _sc_gather(ps.reshape(BATCH * SEQ), table)


# speedup vs baseline: 1.3706x; 1.3601x over previous
import functools

import jax
import jax.numpy as jnp
from jax import lax
from jax.experimental import pallas as pl
from jax.experimental.pallas import tpu as pltpu
from jax.experimental.pallas import tpu_sc as plsc

BATCH = 4096
SEQ = 200
SEQ_PAD = 256
EMBED = 64
CHUNK = 512
NCHUNK = BATCH // CHUNK
HALF = BATCH // 2
PBLK = 128
NPB = HALF // PBLK
NBUF = 4
IDS_T_ROWS = 8


def _permute_body(s_ref, pst_ref, lrow_ref, rank_ref):
    f32 = jnp.float32
    s = s_ref[...]
    mask = (s > 0).astype(f32)
    lcol = jnp.sum(mask, axis=1, keepdims=True)

    i0 = lax.broadcasted_iota(jnp.int32, (CHUNK, CHUNK), 0)
    i1 = lax.broadcasted_iota(jnp.int32, (CHUNK, CHUNK), 1)
    eye = (i0 == i1).astype(f32)
    for c in range(NCHUNK):
        lc = lcol[c * CHUNK:(c + 1) * CHUNK, :]
        lrow_ref[:, c * CHUNK:(c + 1) * CHUNK] = lax.dot_general(
            lc, eye, (((0,), (0,)), ((), ())))
    lrow = lrow_ref[...]

    jj = lax.broadcasted_iota(jnp.int32, (CHUNK, BATCH), 1)
    ii = lax.broadcasted_iota(jnp.int32, (CHUNK, BATCH), 0)
    for c in range(NCHUNK):
        li = lcol[c * CHUNK:(c + 1) * CHUNK, :]
        gi = ii + c * CHUNK
        gt = (lrow > li).astype(f32)
        tie = ((lrow == li) & (jj < gi)).astype(f32)
        rank_ref[c * CHUNK:(c + 1) * CHUNK, :] = jnp.sum(
            gt + tie, axis=1, keepdims=True)

    jjf = jj.astype(f32)
    pst = jnp.zeros((SEQ_PAD, BATCH), f32)
    dims = (((0,), (0,)), ((), ()))
    for c in range(NCHUNK):
        rc = rank_ref[c * CHUNK:(c + 1) * CHUNK, :]
        match = (rc == jjf).astype(f32)
        sc_rows = s[c * CHUNK:(c + 1) * CHUNK, :]
        pa = lax.dot_general((sc_rows >> 16).astype(f32), match, dims)
        pb = lax.dot_general(((sc_rows >> 8) & 255).astype(f32), match, dims)
        pc = lax.dot_general((sc_rows & 255).astype(f32), match, dims)
        pst = pst + (pa * 65536.0 + pb * 256.0 + pc)
    pst_ref[...] = pst[:SEQ, :].astype(jnp.int32)


def _tc_permute(samples_padded):
    return pl.pallas_call(
        _permute_body,
        out_shape=jax.ShapeDtypeStruct((SEQ, BATCH), jnp.int32),
        scratch_shapes=[
            pltpu.VMEM((1, BATCH), jnp.float32),
            pltpu.VMEM((BATCH, 1), jnp.float32),
        ],
    )(samples_padded)


def _sc_gather(ids_t_flat, table):
    info = plsc.get_sparse_core_info()
    nc, ns = info.num_cores, info.num_subcores
    nw = nc * ns
    units = SEQ * NPB * 2
    upw = units // nw
    mesh = plsc.VectorSubcoreMesh(core_axis_name="c", subcore_axis_name="s")

    @functools.partial(
        pl.kernel,
        mesh=mesh,
        out_type=jax.ShapeDtypeStruct((SEQ, HALF, 2 * EMBED), jnp.float32),
        scratch_types=[
            pltpu.VMEM((IDS_T_ROWS * BATCH,), jnp.int32),
            [pltpu.VMEM((PBLK, EMBED), jnp.float32) for _ in range(NBUF)],
            [pltpu.SemaphoreType.DMA for _ in range(NBUF)],
            [pltpu.SemaphoreType.DMA for _ in range(NBUF)],
        ],
        compiler_params=pltpu.CompilerParams(use_tc_tiling_on_sc=False),
    )
    def k(ids_hbm, table_hbm, out_hbm, ids_v, bufs, gsems, wsems):
        wid = lax.axis_index("s") * nc + lax.axis_index("c")
        u0 = wid * upw
        tstart = jnp.minimum((u0 // (2 * NPB)).astype(jnp.int32),
                             SEQ - IDS_T_ROWS)
        pltpu.sync_copy(ids_hbm.at[pl.ds(tstart * BATCH, IDS_T_ROWS * BATCH)],
                        ids_v)

        def unit_coords(h):
            unit = (u0 + h) // 2
            t = unit // NPB
            p0 = (unit % NPB) * PBLK
            side = (u0 + h) % 2
            b0 = p0 + side * HALF
            return t, p0, side, b0

        def issue_gather(h, buf, gsem):
            t, _, _, b0 = unit_coords(h)
            pltpu.async_copy(
                table_hbm.at[ids_v.at[pl.ds((t - tstart) * BATCH + b0, PBLK)]],
                buf, gsem)

        def wait_gather(buf, gsem):
            pltpu.make_async_copy(
                table_hbm.at[ids_v.at[pl.ds(0, PBLK)]], buf, gsem).wait()

        def dst(h):
            t, p0, side, _ = unit_coords(h)
            return out_hbm.at[t, pl.ds(p0, PBLK), pl.ds(side * EMBED, EMBED)]

        def issue_write(h, buf, wsem):
            pltpu.async_copy(buf, dst(h), wsem)

        def wait_write(h, buf, wsem):
            pltpu.make_async_copy(buf, dst(h), wsem).wait()

        for b in range(NBUF):
            issue_gather(b, bufs[b], gsems[b])

        def body(kk, carry):
            h = NBUF * kk
            for b in range(NBUF):
                wait_gather(bufs[b], gsems[b])
                issue_write(h + b, bufs[b], wsems[b])

            for b in range(NBUF):
                @pl.when(h + NBUF + b < upw)
                def _(b=b):
                    wait_write(h + b, bufs[b], wsems[b])
                    issue_gather(h + NBUF + b, bufs[b], gsems[b])

            return carry

        lax.fori_loop(0, upw // NBUF, body, 0)
        for b in range(NBUF):
            wait_write(upw - NBUF + b, bufs[b], wsems[b])

    return k(ids_t_flat, table)


def _fmt_body(x_ref, o_ref):
    f32 = jnp.float32
    i0 = lax.broadcasted_iota(jnp.int32, (PBLK, PBLK), 0)
    i1 = lax.broadcasted_iota(jnp.int32, (PBLK, PBLK), 1)
    eye = (i0 == i1).astype(f32)
    for c in range(NPB):
        xc = x_ref[0, c * PBLK:(c + 1) * PBLK, :]
        xt = lax.dot_general(xc, eye, (((0,), (0,)), ((), ())),
                             precision=lax.Precision.HIGHEST)
        o_ref[0, :, c * PBLK:(c + 1) * PBLK] = xt[:EMBED, :]
        o_ref[0, :, HALF + c * PBLK:HALF + (c + 1) * PBLK] = xt[EMBED:, :]


def _tc_format(f):
    return pl.pallas_call(
        _fmt_body,
        grid=(SEQ,),
        in_specs=[pl.BlockSpec((1, HALF, 2 * EMBED), lambda i: (i, 0, 0))],
        out_specs=pl.BlockSpec((1, EMBED, BATCH), lambda i: (i, 0, 0)),
        out_shape=jax.ShapeDtypeStruct((SEQ, EMBED, BATCH), jnp.float32),
    )(f)


def kernel(samples, table):
    s32 = samples.astype(jnp.int32)
    sp = jnp.pad(s32, ((0, 0), (0, SEQ_PAD - SEQ)))
    ids_t = _tc_permute(sp)
    f = _sc_gather(ids_t.reshape(SEQ * BATCH), table)
    return jnp.transpose(_tc_format(f), (2, 0, 1))


# speedup vs baseline: 1.5531x; 1.1331x over previous
import functools

import jax
import jax.numpy as jnp
from jax import lax
from jax.experimental import pallas as pl
from jax.experimental.pallas import tpu as pltpu
from jax.experimental.pallas import tpu_sc as plsc

BATCH = 4096
SEQ = 200
SEQ_PAD = 256
EMBED = 64
CHUNK = 512
NCHUNK = BATCH // CHUNK
HALF = BATCH // 2
PBLK = 128
NPB = HALF // PBLK
NBUF = 4
IDS_T_ROWS = 8


def _permute_body(s_ref, pst_ref, lrow_ref, rank_ref):
    f32 = jnp.float32
    s = s_ref[...]
    mask = (s > 0).astype(f32)
    lcol = jnp.sum(mask, axis=1, keepdims=True)

    i0 = lax.broadcasted_iota(jnp.int32, (CHUNK, CHUNK), 0)
    i1 = lax.broadcasted_iota(jnp.int32, (CHUNK, CHUNK), 1)
    eye = (i0 == i1).astype(f32)
    for c in range(NCHUNK):
        lc = lcol[c * CHUNK:(c + 1) * CHUNK, :]
        lrow_ref[:, c * CHUNK:(c + 1) * CHUNK] = lax.dot_general(
            lc, eye, (((0,), (0,)), ((), ())))
    lrow = lrow_ref[...]

    jj = lax.broadcasted_iota(jnp.int32, (CHUNK, BATCH), 1)
    ii = lax.broadcasted_iota(jnp.int32, (CHUNK, BATCH), 0)
    for c in range(NCHUNK):
        li = lcol[c * CHUNK:(c + 1) * CHUNK, :]
        gi = ii + c * CHUNK
        gt = (lrow > li).astype(f32)
        tie = ((lrow == li) & (jj < gi)).astype(f32)
        rank_ref[c * CHUNK:(c + 1) * CHUNK, :] = jnp.sum(
            gt + tie, axis=1, keepdims=True)

    jjf = jj.astype(f32)
    pst = jnp.zeros((SEQ_PAD, BATCH), f32)
    dims = (((0,), (0,)), ((), ()))
    for c in range(NCHUNK):
        rc = rank_ref[c * CHUNK:(c + 1) * CHUNK, :]
        match = (rc == jjf).astype(f32)
        sc_rows = s[c * CHUNK:(c + 1) * CHUNK, :]
        pa = lax.dot_general((sc_rows >> 16).astype(f32), match, dims)
        pb = lax.dot_general(((sc_rows >> 8) & 255).astype(f32), match, dims)
        pc = lax.dot_general((sc_rows & 255).astype(f32), match, dims)
        pst = pst + (pa * 65536.0 + pb * 256.0 + pc)
    pst_ref[...] = pst[:SEQ, :].astype(jnp.int32)


def _tc_permute(samples_padded):
    return pl.pallas_call(
        _permute_body,
        out_shape=jax.ShapeDtypeStruct((SEQ, BATCH), jnp.int32),
        scratch_shapes=[
            pltpu.VMEM((1, BATCH), jnp.float32),
            pltpu.VMEM((BATCH, 1), jnp.float32),
        ],
    )(samples_padded)


def _sc_gather(ids_t_flat, table):
    info = plsc.get_sparse_core_info()
    nc, ns = info.num_cores, info.num_subcores
    nw = nc * ns
    units = SEQ * NPB * 2
    upw = units // nw
    mesh = plsc.VectorSubcoreMesh(core_axis_name="c", subcore_axis_name="s")

    @functools.partial(
        pl.kernel,
        mesh=mesh,
        out_type=jax.ShapeDtypeStruct((SEQ, HALF, 2 * EMBED), jnp.float32),
        scratch_types=[
            pltpu.VMEM((IDS_T_ROWS * BATCH,), jnp.int32),
            [pltpu.VMEM((PBLK, EMBED), jnp.float32) for _ in range(NBUF)],
            [pltpu.SemaphoreType.DMA for _ in range(NBUF)],
            [pltpu.SemaphoreType.DMA for _ in range(NBUF)],
        ],
        compiler_params=pltpu.CompilerParams(use_tc_tiling_on_sc=False),
    )
    def k(ids_hbm, table_hbm, out_hbm, ids_v, bufs, gsems, wsems):
        wid = lax.axis_index("s") * nc + lax.axis_index("c")
        u0 = wid * upw
        tstart = jnp.minimum((u0 // (2 * NPB)).astype(jnp.int32),
                             SEQ - IDS_T_ROWS)
        pltpu.sync_copy(ids_hbm.at[pl.ds(tstart * BATCH, IDS_T_ROWS * BATCH)],
                        ids_v)

        def unit_coords(h):
            unit = (u0 + h) // 2
            t = unit // NPB
            p0 = (unit % NPB) * PBLK
            side = (u0 + h) % 2
            b0 = p0 + side * HALF
            return t, p0, side, b0

        def issue_gather(h, buf, gsem):
            t, _, _, b0 = unit_coords(h)
            pltpu.async_copy(
                table_hbm.at[ids_v.at[pl.ds((t - tstart) * BATCH + b0, PBLK)]],
                buf, gsem)

        def wait_gather(buf, gsem):
            pltpu.make_async_copy(
                table_hbm.at[ids_v.at[pl.ds(0, PBLK)]], buf, gsem).wait()

        def dst(h):
            t, p0, side, _ = unit_coords(h)
            return out_hbm.at[t, pl.ds(p0, PBLK), pl.ds(side * EMBED, EMBED)]

        def issue_write(h, buf, wsem):
            pltpu.async_copy(buf, dst(h), wsem)

        def wait_write(h, buf, wsem):
            pltpu.make_async_copy(buf, dst(h), wsem).wait()

        for b in range(NBUF):
            issue_gather(b, bufs[b], gsems[b])

        def body(kk, carry):
            h = NBUF * kk
            for b in range(NBUF):
                wait_gather(bufs[b], gsems[b])
                issue_write(h + b, bufs[b], wsems[b])

            for b in range(NBUF):
                @pl.when(h + NBUF + b < upw)
                def _(b=b):
                    wait_write(h + b, bufs[b], wsems[b])
                    issue_gather(h + NBUF + b, bufs[b], gsems[b])

            return carry

        lax.fori_loop(0, upw // NBUF, body, 0)
        for b in range(NBUF):
            wait_write(upw - NBUF + b, bufs[b], wsems[b])

    return k(ids_t_flat, table)


def _fmt_body(x_ref, o_ref):
    f32 = jnp.float32
    i0 = lax.broadcasted_iota(jnp.int32, (PBLK, PBLK), 0)
    i1 = lax.broadcasted_iota(jnp.int32, (PBLK, PBLK), 1)
    eye = (i0 == i1).astype(f32)
    for c in range(NPB):
        xc = x_ref[0, c * PBLK:(c + 1) * PBLK, :]
        xt = jnp.transpose(xc)
        o_ref[0, :, c * PBLK:(c + 1) * PBLK] = xt[:EMBED, :]
        o_ref[0, :, HALF + c * PBLK:HALF + (c + 1) * PBLK] = xt[EMBED:, :]


def _tc_format(f):
    return pl.pallas_call(
        _fmt_body,
        grid=(SEQ,),
        in_specs=[pl.BlockSpec((1, HALF, 2 * EMBED), lambda i: (i, 0, 0))],
        out_specs=pl.BlockSpec((1, EMBED, BATCH), lambda i: (i, 0, 0)),
        out_shape=jax.ShapeDtypeStruct((SEQ, EMBED, BATCH), jnp.float32),
    )(f)


def kernel(samples, table):
    s32 = samples.astype(jnp.int32)
    sp = jnp.pad(s32, ((0, 0), (0, SEQ_PAD - SEQ)))
    ids_t = _tc_permute(sp)
    f = _sc_gather(ids_t.reshape(SEQ * BATCH), table)
    return jnp.transpose(_tc_format(f), (2, 0, 1))


# speedup vs baseline: 1.5585x; 1.0035x over previous
import functools

import jax
import jax.numpy as jnp
from jax import lax
from jax.experimental import pallas as pl
from jax.experimental.pallas import tpu as pltpu
from jax.experimental.pallas import tpu_sc as plsc

BATCH = 4096
SEQ = 200
SEQ_PAD = 256
EMBED = 64
CHUNK = 512
NCHUNK = BATCH // CHUNK
HALF = BATCH // 2
PBLK = 128
NPB = HALF // PBLK
NBUF = 4
IDS_T_ROWS = 8


def _permute_body(s_ref, pst_ref, lrow_ref, rank_ref):
    f32 = jnp.float32
    s = s_ref[...]
    mask = (s > 0).astype(f32)
    lcol = jnp.sum(mask, axis=1, keepdims=True)

    i0 = lax.broadcasted_iota(jnp.int32, (CHUNK, CHUNK), 0)
    i1 = lax.broadcasted_iota(jnp.int32, (CHUNK, CHUNK), 1)
    eye = (i0 == i1).astype(f32)
    for c in range(NCHUNK):
        lc = lcol[c * CHUNK:(c + 1) * CHUNK, :]
        lrow_ref[:, c * CHUNK:(c + 1) * CHUNK] = lax.dot_general(
            lc, eye, (((0,), (0,)), ((), ())))
    lrow = lrow_ref[...]

    jj = lax.broadcasted_iota(jnp.int32, (CHUNK, BATCH), 1)
    ii = lax.broadcasted_iota(jnp.int32, (CHUNK, BATCH), 0)
    for c in range(NCHUNK):
        li = lcol[c * CHUNK:(c + 1) * CHUNK, :]
        gi = ii + c * CHUNK
        gt = (lrow > li).astype(f32)
        tie = ((lrow == li) & (jj < gi)).astype(f32)
        rank_ref[c * CHUNK:(c + 1) * CHUNK, :] = jnp.sum(
            gt + tie, axis=1, keepdims=True)

    jjf = jj.astype(f32)
    pst = jnp.zeros((SEQ_PAD, BATCH), f32)
    dims = (((0,), (0,)), ((), ()))
    for c in range(NCHUNK):
        rc = rank_ref[c * CHUNK:(c + 1) * CHUNK, :]
        match = (rc == jjf).astype(f32)
        sc_rows = s[c * CHUNK:(c + 1) * CHUNK, :]
        pa = lax.dot_general((sc_rows >> 16).astype(f32), match, dims)
        pb = lax.dot_general(((sc_rows >> 8) & 255).astype(f32), match, dims)
        pc = lax.dot_general((sc_rows & 255).astype(f32), match, dims)
        pst = pst + (pa * 65536.0 + pb * 256.0 + pc)
    pst_ref[...] = pst[:SEQ, :].astype(jnp.int32)


def _tc_permute(samples_padded):
    return pl.pallas_call(
        _permute_body,
        out_shape=jax.ShapeDtypeStruct((SEQ, BATCH), jnp.int32),
        scratch_shapes=[
            pltpu.VMEM((1, BATCH), jnp.float32),
            pltpu.VMEM((BATCH, 1), jnp.float32),
        ],
    )(samples_padded)


def _sc_gather(ids_t_flat, table):
    info = plsc.get_sparse_core_info()
    nc, ns = info.num_cores, info.num_subcores
    nw = nc * ns
    units = SEQ * NPB * 2
    upw = units // nw
    mesh = plsc.VectorSubcoreMesh(core_axis_name="c", subcore_axis_name="s")

    @functools.partial(
        pl.kernel,
        mesh=mesh,
        out_type=jax.ShapeDtypeStruct((SEQ, HALF, 2 * EMBED), jnp.float32),
        scratch_types=[
            pltpu.VMEM((IDS_T_ROWS * BATCH,), jnp.int32),
            [pltpu.VMEM((PBLK, EMBED), jnp.float32) for _ in range(NBUF)],
            [pltpu.SemaphoreType.DMA for _ in range(NBUF)],
            [pltpu.SemaphoreType.DMA for _ in range(NBUF)],
        ],
        compiler_params=pltpu.CompilerParams(use_tc_tiling_on_sc=False),
    )
    def k(ids_hbm, table_hbm, out_hbm, ids_v, bufs, gsems, wsems):
        wid = lax.axis_index("s") * nc + lax.axis_index("c")
        u0 = wid * upw
        tstart = jnp.minimum((u0 // (2 * NPB)).astype(jnp.int32),
                             SEQ - IDS_T_ROWS)
        pltpu.sync_copy(ids_hbm.at[pl.ds(tstart * BATCH, IDS_T_ROWS * BATCH)],
                        ids_v)

        def unit_coords(h):
            unit = (u0 + h) // 2
            t = unit // NPB
            p0 = (unit % NPB) * PBLK
            side = (u0 + h) % 2
            b0 = p0 + side * HALF
            return t, p0, side, b0

        def issue_gather(h, buf, gsem):
            t, _, _, b0 = unit_coords(h)
            pltpu.async_copy(
                table_hbm.at[ids_v.at[pl.ds((t - tstart) * BATCH + b0, PBLK)]],
                buf, gsem)

        def wait_gather(buf, gsem):
            pltpu.make_async_copy(
                table_hbm.at[ids_v.at[pl.ds(0, PBLK)]], buf, gsem).wait()

        def dst(h):
            t, p0, side, _ = unit_coords(h)
            return out_hbm.at[t, pl.ds(p0, PBLK), pl.ds(side * EMBED, EMBED)]

        def issue_write(h, buf, wsem):
            pltpu.async_copy(buf, dst(h), wsem)

        def wait_write(h, buf, wsem):
            pltpu.make_async_copy(buf, dst(h), wsem).wait()

        for b in range(NBUF):
            issue_gather(b, bufs[b], gsems[b])

        def body(kk, carry):
            h = NBUF * kk
            for b in range(NBUF):
                wait_gather(bufs[b], gsems[b])
                issue_write(h + b, bufs[b], wsems[b])

            for b in range(NBUF):
                @pl.when(h + NBUF + b < upw)
                def _(b=b):
                    wait_write(h + b, bufs[b], wsems[b])
                    issue_gather(h + NBUF + b, bufs[b], gsems[b])

            return carry

        lax.fori_loop(0, upw // NBUF, body, 0)
        for b in range(NBUF):
            wait_write(upw - NBUF + b, bufs[b], wsems[b])

    return k(ids_t_flat, table)


def _fmt_body(x_ref, o_ref):
    for c in range(NPB):
        xc = x_ref[0, c * PBLK:(c + 1) * PBLK, :]
        xt = jnp.transpose(xc)
        o_ref[0, :, c * PBLK:(c + 1) * PBLK] = xt[:EMBED, :]
        o_ref[0, :, HALF + c * PBLK:HALF + (c + 1) * PBLK] = xt[EMBED:, :]


def _tc_format(f):
    return pl.pallas_call(
        _fmt_body,
        grid=(SEQ,),
        in_specs=[pl.BlockSpec((1, HALF, 2 * EMBED), lambda i: (i, 0, 0))],
        out_specs=pl.BlockSpec((1, EMBED, BATCH), lambda i: (i, 0, 0)),
        out_shape=jax.ShapeDtypeStruct((SEQ, EMBED, BATCH), jnp.float32),
    )(f)


def kernel(samples, table):
    s32 = samples.astype(jnp.int32)
    sp = jnp.pad(s32, ((0, 0), (0, SEQ_PAD - SEQ)))
    ids_t = _tc_permute(sp)
    f = _sc_gather(ids_t.reshape(SEQ * BATCH), table)
    return jnp.transpose(_tc_format(f), (2, 0, 1))
